# baseline (speedup 1.0000x reference)
import functools

import jax
import jax.numpy as jnp
from jax import lax
from jax.experimental import pallas as pl
from jax.experimental.pallas import tpu as pltpu
from jax.experimental.pallas import tpu_sc as plsc

B = 1024
D = 64
G = 8
BLK = 128
GAMMA = 1e-10

_info = plsc.get_sparse_core_info()
_NC, _NS, _L = _info.num_cores, _info.num_subcores, _info.num_lanes
_NW = _NC * _NS
_BPW = B // _NW

_sc_mesh = plsc.VectorSubcoreMesh(core_axis_name="c", subcore_axis_name="s")


@functools.partial(
    pl.kernel,
    mesh=_sc_mesh,
    out_type=[
        jax.ShapeDtypeStruct((G, BLK, D), jnp.float32),
        jax.ShapeDtypeStruct((G, BLK, D), jnp.float32),
        jax.ShapeDtypeStruct((G, BLK, D), jnp.float32),
    ],
    scratch_types=[
        pltpu.VMEM((_BPW,), jnp.int32),
        pltpu.VMEM((_BPW,), jnp.int32),
        pltpu.VMEM((_BPW,), jnp.int32),
        pltpu.VMEM((_BPW, G, D), jnp.float32),
        pltpu.VMEM((_BPW, G, D), jnp.float32),
        pltpu.VMEM((_BPW, G, D), jnp.float32),
        pltpu.VMEM((_BPW, D), jnp.float32),
        pltpu.SemaphoreType.DMA,
        pltpu.SemaphoreType.DMA,
        pltpu.SemaphoreType.DMA,
    ],
)
def _gather3(user_tab, item_tab, users_h, items_h, neg_h,
             u_out, p_out, n_out,
             idx_u, idx_p, idx_n, rows_u, rows_p, rows_n, sel_v,
             sem_u, sem_p, sem_n):
    wid = lax.axis_index("s") * _NC + lax.axis_index("c")
    base = wid * _BPW
    sl = pl.ds(base, _BPW)
    pltpu.sync_copy(users_h.at[sl], idx_u)
    pltpu.sync_copy(items_h.at[sl], idx_p)
    pltpu.sync_copy(neg_h.at[sl], idx_n)
    plan = ((idx_u, user_tab, rows_u, sem_u, u_out),
            (idx_p, item_tab, rows_p, sem_p, p_out),
            (idx_n, item_tab, rows_n, sem_n, n_out))
    raw_chunks = {}
    copies = {0: [], 1: [], 2: []}
    for t, (idx_v, tab, rows_v, sem, _) in enumerate(plan):
        for c in range(_BPW // _L):
            raw = idx_v[pl.ds(c * _L, _L)]
            raw_chunks[(t, c)] = raw
            g_chunk = lax.shift_right_logical(raw, 3)
            for l in range(_L):
                copies[t].append(
                    pltpu.async_copy(tab.at[g_chunk[l]],
                                     rows_v.at[c * _L + l], sem))
    oa = lax.shift_right_logical(wid, 2)
    ob = (wid & 3) * _BPW
    for t, (idx_v, tab, rows_v, sem, out) in enumerate(plan):
        for cp in copies[t]:
            cp.wait()
        for c in range(_BPW // _L):
            raw = raw_chunks[(t, c)]
            for l in range(_L):
                k = c * _L + l
                rk = raw[l] & 7
                for q in range(D // _L):
                    qs = pl.ds(q * _L, _L)
                    sel_v[k, qs] = rows_v[k, rk, qs]
        pltpu.sync_copy(sel_v, out.at[oa, pl.ds(ob, _BPW)])


_RPB = 500


def _repack_body(in_ref, out_ref):
    out_ref[...] = in_ref[...].reshape(_RPB, G, D)


_repack_call = pl.pallas_call(
    _repack_body,
    grid=(12500 // _RPB,),
    in_specs=[pl.BlockSpec((_RPB * G, D), lambda i: (i, 0))],
    out_specs=pl.BlockSpec((_RPB, G, D), lambda i: (i, 0, 0)),
    out_shape=jax.ShapeDtypeStruct((12500, G, D), jnp.float32),
)


def _loss_body(gu_all_ref, gu_blk_ref, gp_ref, gn_ref, out_ref):
    i = pl.program_id(0)

    @pl.when(i == 0)
    def _init():
        out_ref[0, 0] = 0.0

    u_all = gu_all_ref[...].reshape(B, D)
    u_blk = gu_blk_ref[...].reshape(BLK, D)
    p = gp_ref[...].reshape(BLK, D)
    n = gn_ref[...].reshape(BLK, D)
    pos = jnp.sum(u_blk * p, axis=1, keepdims=True)
    neg = lax.dot_general(n, u_all,
                          (((1,), (1,)), ((), ())),
                          preferred_element_type=jnp.float32)
    x = pos - neg
    loss = -jnp.log(GAMMA + jax.nn.sigmoid(x))
    out_ref[0, 0] += jnp.sum(loss) * (1.0 / (B * B))


_loss_call = pl.pallas_call(
    _loss_body,
    grid=(B // BLK,),
    in_specs=[
        pl.BlockSpec((G, BLK, D), lambda i: (0, 0, 0)),
        pl.BlockSpec((1, BLK, D), lambda i: (i, 0, 0)),
        pl.BlockSpec((1, BLK, D), lambda i: (i, 0, 0)),
        pl.BlockSpec((1, BLK, D), lambda i: (i, 0, 0)),
    ],
    out_specs=pl.BlockSpec((1, 1), lambda i: (0, 0), memory_space=pltpu.SMEM),
    out_shape=jax.ShapeDtypeStruct((1, 1), jnp.float32),
)


def kernel(users, items, neg_items, user_table, item_table):
    users = users.astype(jnp.int32)
    items = items.astype(jnp.int32)
    neg = neg_items.reshape(-1).astype(jnp.int32)
    ut3 = _repack_call(user_table)
    it3 = item_table.reshape(-1, G, D)
    g_u, g_p, g_n = _gather3(ut3, it3, users, items, neg)
    out = _loss_call(g_u, g_u, g_p, g_n)
    return out[0, 0]


# speedup vs baseline: 1.3974x; 1.3974x over previous
import functools

import jax
import jax.numpy as jnp
from jax import lax
from jax.experimental import pallas as pl
from jax.experimental.pallas import tpu as pltpu
from jax.experimental.pallas import tpu_sc as plsc

B = 1024
D = 64
G = 8
BLK = 128
GAMMA = 1e-10

_info = plsc.get_sparse_core_info()
_NC, _NS, _L = _info.num_cores, _info.num_subcores, _info.num_lanes
_NW = _NC * _NS
_BPW = B // _NW

_sc_mesh = plsc.VectorSubcoreMesh(core_axis_name="c", subcore_axis_name="s")


@functools.partial(
    pl.kernel,
    mesh=_sc_mesh,
    out_type=[
        jax.ShapeDtypeStruct((G, BLK, D), jnp.float32),
        jax.ShapeDtypeStruct((G, BLK, D), jnp.float32),
        jax.ShapeDtypeStruct((G, BLK, D), jnp.float32),
    ],
    scratch_types=[
        pltpu.VMEM((_BPW,), jnp.int32),
        pltpu.VMEM((_BPW,), jnp.int32),
        pltpu.VMEM((_BPW,), jnp.int32),
        pltpu.VMEM((_BPW, G, D), jnp.float32),
        pltpu.VMEM((_BPW, G, D), jnp.float32),
        pltpu.VMEM((_BPW, G, D), jnp.float32),
        pltpu.VMEM((_BPW, D), jnp.float32),
        pltpu.SemaphoreType.DMA,
        pltpu.SemaphoreType.DMA,
        pltpu.SemaphoreType.DMA,
    ],
)
def _gather3(user_tab, item_tab, users_h, items_h, neg_h,
             u_out, p_out, n_out,
             idx_u, idx_p, idx_n, rows_u, rows_p, rows_n, sel_v,
             sem_u, sem_p, sem_n):
    wid = lax.axis_index("s") * _NC + lax.axis_index("c")
    base = wid * _BPW
    sl = pl.ds(base, _BPW)
    pltpu.sync_copy(users_h.at[sl], idx_u)
    pltpu.sync_copy(items_h.at[sl], idx_p)
    pltpu.sync_copy(neg_h.at[sl], idx_n)
    plan = ((idx_u, user_tab, rows_u, sem_u, u_out),
            (idx_p, item_tab, rows_p, sem_p, p_out),
            (idx_n, item_tab, rows_n, sem_n, n_out))
    raw_chunks = {}
    copies = {0: [], 1: [], 2: []}
    for t, (idx_v, tab, rows_v, sem, _) in enumerate(plan):
        for c in range(_BPW // _L):
            raw = idx_v[pl.ds(c * _L, _L)]
            raw_chunks[(t, c)] = raw
            g_chunk = lax.shift_right_logical(raw, 3)
            for l in range(_L):
                copies[t].append(
                    pltpu.async_copy(tab.at[g_chunk[l]],
                                     rows_v.at[c * _L + l], sem))
    oa = lax.shift_right_logical(wid, 2)
    ob = (wid & 3) * _BPW
    for t, (idx_v, tab, rows_v, sem, out) in enumerate(plan):
        for cp in copies[t]:
            cp.wait()
        for c in range(_BPW // _L):
            raw = raw_chunks[(t, c)]
            for l in range(_L):
                k = c * _L + l
                rk = raw[l] & 7
                for q in range(D // _L):
                    qs = pl.ds(q * _L, _L)
                    sel_v[k, qs] = rows_v[k, rk, qs]
        pltpu.sync_copy(sel_v, out.at[oa, pl.ds(ob, _BPW)])


def _loss_body(gu_all_ref, gu_blk_ref, gp_ref, gn_ref, out_ref):
    i = pl.program_id(0)

    @pl.when(i == 0)
    def _init():
        out_ref[0, 0] = 0.0

    u_all = gu_all_ref[...].reshape(B, D)
    u_blk = gu_blk_ref[...].reshape(BLK, D)
    p = gp_ref[...].reshape(BLK, D)
    n = gn_ref[...].reshape(BLK, D)
    pos = jnp.sum(u_blk * p, axis=1, keepdims=True)
    neg = lax.dot_general(n, u_all,
                          (((1,), (1,)), ((), ())),
                          preferred_element_type=jnp.float32)
    x = pos - neg
    loss = -jnp.log(GAMMA + jax.nn.sigmoid(x))
    out_ref[0, 0] += jnp.sum(loss) * (1.0 / (B * B))


_loss_call = pl.pallas_call(
    _loss_body,
    grid=(B // BLK,),
    in_specs=[
        pl.BlockSpec((G, BLK, D), lambda i: (0, 0, 0)),
        pl.BlockSpec((1, BLK, D), lambda i: (i, 0, 0)),
        pl.BlockSpec((1, BLK, D), lambda i: (i, 0, 0)),
        pl.BlockSpec((1, BLK, D), lambda i: (i, 0, 0)),
    ],
    out_specs=pl.BlockSpec((1, 1), lambda i: (0, 0), memory_space=pltpu.SMEM),
    out_shape=jax.ShapeDtypeStruct((1, 1), jnp.float32),
)


def kernel(users, items, neg_items, user_table, item_table):
    users = users.astype(jnp.int32)
    items = items.astype(jnp.int32)
    neg = neg_items.reshape(-1).astype(jnp.int32)
    ut3 = user_table.reshape(-1, G, D)
    it3 = item_table.reshape(-1, G, D)
    g_u, g_p, g_n = _gather3(ut3, it3, users, items, neg)
    out = _loss_call(g_u, g_u, g_p, g_n)
    return out[0, 0]


# speedup vs baseline: 1.4354x; 1.0272x over previous
import functools

import jax
import jax.numpy as jnp
from jax import lax
from jax.experimental import pallas as pl
from jax.experimental.pallas import tpu as pltpu
from jax.experimental.pallas import tpu_sc as plsc

B = 1024
D = 64
G = 8
BLK = 256
GAMMA = 1e-10

_info = plsc.get_sparse_core_info()
_NC, _NS, _L = _info.num_cores, _info.num_subcores, _info.num_lanes
_NW = _NC * _NS
_BPW = B // _NW

_sc_mesh = plsc.VectorSubcoreMesh(core_axis_name="c", subcore_axis_name="s")


@functools.partial(
    pl.kernel,
    mesh=_sc_mesh,
    out_type=[
        jax.ShapeDtypeStruct((G, 128, D), jnp.float32),
        jax.ShapeDtypeStruct((G, 128, D), jnp.float32),
        jax.ShapeDtypeStruct((G, 128, D), jnp.float32),
    ],
    scratch_types=[
        pltpu.VMEM((_BPW,), jnp.int32),
        pltpu.VMEM((_BPW,), jnp.int32),
        pltpu.VMEM((_BPW,), jnp.int32),
        pltpu.VMEM((_BPW, G, D), jnp.float32),
        pltpu.VMEM((_BPW, G, D), jnp.float32),
        pltpu.VMEM((_BPW, G, D), jnp.float32),
        pltpu.VMEM((_BPW, D), jnp.float32),
        pltpu.SemaphoreType.DMA,
        pltpu.SemaphoreType.DMA,
        pltpu.SemaphoreType.DMA,
    ],
)
def _gather3(user_tab, item_tab, users_h, items_h, neg_h,
             u_out, p_out, n_out,
             idx_u, idx_p, idx_n, rows_u, rows_p, rows_n, sel_v,
             sem_u, sem_p, sem_n):
    wid = lax.axis_index("s") * _NC + lax.axis_index("c")
    base = wid * _BPW
    sl = pl.ds(base, _BPW)
    pltpu.sync_copy(users_h.at[sl], idx_u)
    pltpu.sync_copy(items_h.at[sl], idx_p)
    pltpu.sync_copy(neg_h.at[sl], idx_n)
    plan = ((idx_u, user_tab, rows_u, sem_u, u_out),
            (idx_p, item_tab, rows_p, sem_p, p_out),
            (idx_n, item_tab, rows_n, sem_n, n_out))
    raw_chunks = {}
    copies = {0: [], 1: [], 2: []}
    for t, (idx_v, tab, rows_v, sem, _) in enumerate(plan):
        for c in range(_BPW // _L):
            raw = idx_v[pl.ds(c * _L, _L)]
            raw_chunks[(t, c)] = raw
            g_chunk = lax.shift_right_logical(raw, 3)
            for l in range(_L):
                copies[t].append(
                    pltpu.async_copy(tab.at[g_chunk[l]],
                                     rows_v.at[c * _L + l], sem))
    oa = lax.shift_right_logical(wid, 2)
    ob = (wid & 3) * _BPW
    for t, (idx_v, tab, rows_v, sem, out) in enumerate(plan):
        for cp in copies[t]:
            cp.wait()
        for c in range(_BPW // _L):
            raw = raw_chunks[(t, c)]
            for l in range(_L):
                k = c * _L + l
                rk = raw[l] & 7
                for q in range(D // _L):
                    qs = pl.ds(q * _L, _L)
                    sel_v[k, qs] = rows_v[k, rk, qs]
        pltpu.sync_copy(sel_v, out.at[oa, pl.ds(ob, _BPW)])


def _loss_body(gu_all_ref, gu_blk_ref, gp_ref, gn_ref, out_ref):
    i = pl.program_id(0)

    @pl.when(i == 0)
    def _init():
        out_ref[0, 0] = 0.0

    u_all = gu_all_ref[...].reshape(B, D)
    u_blk = gu_blk_ref[...].reshape(BLK, D)
    p = gp_ref[...].reshape(BLK, D)
    n = gn_ref[...].reshape(BLK, D)
    pos = jnp.sum(u_blk * p, axis=1, keepdims=True)
    neg = lax.dot_general(n, u_all,
                          (((1,), (1,)), ((), ())),
                          preferred_element_type=jnp.float32)
    x = pos - neg
    loss = -jnp.log(GAMMA + jax.nn.sigmoid(x))
    out_ref[0, 0] += jnp.sum(loss) * (1.0 / (B * B))


_loss_call = pl.pallas_call(
    _loss_body,
    grid=(B // BLK,),
    in_specs=[
        pl.BlockSpec((G, 128, D), lambda i: (0, 0, 0)),
        pl.BlockSpec((BLK // 128, 128, D), lambda i: (i, 0, 0)),
        pl.BlockSpec((BLK // 128, 128, D), lambda i: (i, 0, 0)),
        pl.BlockSpec((BLK // 128, 128, D), lambda i: (i, 0, 0)),
    ],
    out_specs=pl.BlockSpec((1, 1), lambda i: (0, 0), memory_space=pltpu.SMEM),
    out_shape=jax.ShapeDtypeStruct((1, 1), jnp.float32),
)


def kernel(users, items, neg_items, user_table, item_table):
    users = users.astype(jnp.int32)
    items = items.astype(jnp.int32)
    neg = neg_items.reshape(-1).astype(jnp.int32)
    ut3 = user_table.reshape(-1, G, D)
    it3 = item_table.reshape(-1, G, D)
    g_u, g_p, g_n = _gather3(ut3, it3, users, items, neg)
    out = _loss_call(g_u, g_u, g_p, g_n)
    return out[0, 0]


# speedup vs baseline: 1.4516x; 1.0113x over previous
import functools

import jax
import jax.numpy as jnp
from jax import lax
from jax.experimental import pallas as pl
from jax.experimental.pallas import tpu as pltpu
from jax.experimental.pallas import tpu_sc as plsc

B = 1024
D = 64
G = 8
BLK = 512
GAMMA = 1e-10

_info = plsc.get_sparse_core_info()
_NC, _NS, _L = _info.num_cores, _info.num_subcores, _info.num_lanes
_NW = _NC * _NS
_BPW = B // _NW

_sc_mesh = plsc.VectorSubcoreMesh(core_axis_name="c", subcore_axis_name="s")


@functools.partial(
    pl.kernel,
    mesh=_sc_mesh,
    out_type=[
        jax.ShapeDtypeStruct((G, 128, D), jnp.float32),
        jax.ShapeDtypeStruct((G, 128, D), jnp.float32),
        jax.ShapeDtypeStruct((G, 128, D), jnp.float32),
    ],
    scratch_types=[
        pltpu.VMEM((_BPW,), jnp.int32),
        pltpu.VMEM((_BPW,), jnp.int32),
        pltpu.VMEM((_BPW,), jnp.int32),
        pltpu.VMEM((_BPW, G, D), jnp.float32),
        pltpu.VMEM((_BPW, G, D), jnp.float32),
        pltpu.VMEM((_BPW, G, D), jnp.float32),
        pltpu.VMEM((_BPW, D), jnp.float32),
        pltpu.SemaphoreType.DMA,
        pltpu.SemaphoreType.DMA,
        pltpu.SemaphoreType.DMA,
    ],
)
def _gather3(user_tab, item_tab, users_h, items_h, neg_h,
             u_out, p_out, n_out,
             idx_u, idx_p, idx_n, rows_u, rows_p, rows_n, sel_v,
             sem_u, sem_p, sem_n):
    wid = lax.axis_index("s") * _NC + lax.axis_index("c")
    base = wid * _BPW
    sl = pl.ds(base, _BPW)
    pltpu.sync_copy(users_h.at[sl], idx_u)
    pltpu.sync_copy(items_h.at[sl], idx_p)
    pltpu.sync_copy(neg_h.at[sl], idx_n)
    plan = ((idx_u, user_tab, rows_u, sem_u, u_out),
            (idx_p, item_tab, rows_p, sem_p, p_out),
            (idx_n, item_tab, rows_n, sem_n, n_out))
    raw_chunks = {}
    copies = {0: [], 1: [], 2: []}
    for t, (idx_v, tab, rows_v, sem, _) in enumerate(plan):
        for c in range(_BPW // _L):
            raw = idx_v[pl.ds(c * _L, _L)]
            raw_chunks[(t, c)] = raw
            g_chunk = lax.shift_right_logical(raw, 3)
            for l in range(_L):
                copies[t].append(
                    pltpu.async_copy(tab.at[g_chunk[l]],
                                     rows_v.at[c * _L + l], sem))
    oa = lax.shift_right_logical(wid, 2)
    ob = (wid & 3) * _BPW
    for t, (idx_v, tab, rows_v, sem, out) in enumerate(plan):
        for cp in copies[t]:
            cp.wait()
        for c in range(_BPW // _L):
            raw = raw_chunks[(t, c)]
            for l in range(_L):
                k = c * _L + l
                rk = raw[l] & 7
                for q in range(D // _L):
                    qs = pl.ds(q * _L, _L)
                    sel_v[k, qs] = rows_v[k, rk, qs]
        pltpu.sync_copy(sel_v, out.at[oa, pl.ds(ob, _BPW)])


def _loss_body(gu_all_ref, gu_blk_ref, gp_ref, gn_ref, out_ref):
    i = pl.program_id(0)

    @pl.when(i == 0)
    def _init():
        out_ref[0, 0] = 0.0

    u_all = gu_all_ref[...].reshape(B, D)
    u_blk = gu_blk_ref[...].reshape(BLK, D)
    p = gp_ref[...].reshape(BLK, D)
    n = gn_ref[...].reshape(BLK, D)
    pos = jnp.sum(u_blk * p, axis=1, keepdims=True)
    neg = lax.dot_general(n, u_all,
                          (((1,), (1,)), ((), ())),
                          preferred_element_type=jnp.float32)
    x = pos - neg
    loss = -jnp.log(GAMMA + jax.nn.sigmoid(x))
    out_ref[0, 0] += jnp.sum(loss) * (1.0 / (B * B))


_loss_call = pl.pallas_call(
    _loss_body,
    grid=(B // BLK,),
    in_specs=[
        pl.BlockSpec((G, 128, D), lambda i: (0, 0, 0)),
        pl.BlockSpec((BLK // 128, 128, D), lambda i: (i, 0, 0)),
        pl.BlockSpec((BLK // 128, 128, D), lambda i: (i, 0, 0)),
        pl.BlockSpec((BLK // 128, 128, D), lambda i: (i, 0, 0)),
    ],
    out_specs=pl.BlockSpec((1, 1), lambda i: (0, 0), memory_space=pltpu.SMEM),
    out_shape=jax.ShapeDtypeStruct((1, 1), jnp.float32),
)


def kernel(users, items, neg_items, user_table, item_table):
    users = users.astype(jnp.int32)
    items = items.astype(jnp.int32)
    neg = neg_items.reshape(-1).astype(jnp.int32)
    ut3 = user_table.reshape(-1, G, D)
    it3 = item_table.reshape(-1, G, D)
    g_u, g_p, g_n = _gather3(ut3, it3, users, items, neg)
    out = _loss_call(g_u, g_u, g_p, g_n)
    return out[0, 0]


# speedup vs baseline: 1.4593x; 1.0053x over previous
import functools

import jax
import jax.numpy as jnp
from jax import lax
from jax.experimental import pallas as pl
from jax.experimental.pallas import tpu as pltpu
from jax.experimental.pallas import tpu_sc as plsc

B = 1024
D = 64
G = 8
BLK = 1024
GAMMA = 1e-10

_info = plsc.get_sparse_core_info()
_NC, _NS, _L = _info.num_cores, _info.num_subcores, _info.num_lanes
_NW = _NC * _NS
_BPW = B // _NW

_sc_mesh = plsc.VectorSubcoreMesh(core_axis_name="c", subcore_axis_name="s")


@functools.partial(
    pl.kernel,
    mesh=_sc_mesh,
    out_type=[
        jax.ShapeDtypeStruct((G, 128, D), jnp.float32),
        jax.ShapeDtypeStruct((G, 128, D), jnp.float32),
        jax.ShapeDtypeStruct((G, 128, D), jnp.float32),
    ],
    scratch_types=[
        pltpu.VMEM((_BPW,), jnp.int32),
        pltpu.VMEM((_BPW,), jnp.int32),
        pltpu.VMEM((_BPW,), jnp.int32),
        pltpu.VMEM((_BPW, G, D), jnp.float32),
        pltpu.VMEM((_BPW, G, D), jnp.float32),
        pltpu.VMEM((_BPW, G, D), jnp.float32),
        pltpu.VMEM((_BPW, D), jnp.float32),
        pltpu.SemaphoreType.DMA,
        pltpu.SemaphoreType.DMA,
        pltpu.SemaphoreType.DMA,
    ],
)
def _gather3(user_tab, item_tab, users_h, items_h, neg_h,
             u_out, p_out, n_out,
             idx_u, idx_p, idx_n, rows_u, rows_p, rows_n, sel_v,
             sem_u, sem_p, sem_n):
    wid = lax.axis_index("s") * _NC + lax.axis_index("c")
    base = wid * _BPW
    sl = pl.ds(base, _BPW)
    pltpu.sync_copy(users_h.at[sl], idx_u)
    pltpu.sync_copy(items_h.at[sl], idx_p)
    pltpu.sync_copy(neg_h.at[sl], idx_n)
    plan = ((idx_u, user_tab, rows_u, sem_u, u_out),
            (idx_p, item_tab, rows_p, sem_p, p_out),
            (idx_n, item_tab, rows_n, sem_n, n_out))
    raw_chunks = {}
    copies = {0: [], 1: [], 2: []}
    for t, (idx_v, tab, rows_v, sem, _) in enumerate(plan):
        for c in range(_BPW // _L):
            raw = idx_v[pl.ds(c * _L, _L)]
            raw_chunks[(t, c)] = raw
            g_chunk = lax.shift_right_logical(raw, 3)
            for l in range(_L):
                copies[t].append(
                    pltpu.async_copy(tab.at[g_chunk[l]],
                                     rows_v.at[c * _L + l], sem))
    oa = lax.shift_right_logical(wid, 2)
    ob = (wid & 3) * _BPW
    for t, (idx_v, tab, rows_v, sem, out) in enumerate(plan):
        for cp in copies[t]:
            cp.wait()
        for c in range(_BPW // _L):
            raw = raw_chunks[(t, c)]
            for l in range(_L):
                k = c * _L + l
                rk = raw[l] & 7
                for q in range(D // _L):
                    qs = pl.ds(q * _L, _L)
                    sel_v[k, qs] = rows_v[k, rk, qs]
        pltpu.sync_copy(sel_v, out.at[oa, pl.ds(ob, _BPW)])


def _loss_body(gu_all_ref, gu_blk_ref, gp_ref, gn_ref, out_ref):
    i = pl.program_id(0)

    @pl.when(i == 0)
    def _init():
        out_ref[0, 0] = 0.0

    u_all = gu_all_ref[...].reshape(B, D)
    u_blk = gu_blk_ref[...].reshape(BLK, D)
    p = gp_ref[...].reshape(BLK, D)
    n = gn_ref[...].reshape(BLK, D)
    pos = jnp.sum(u_blk * p, axis=1, keepdims=True)
    neg = lax.dot_general(n, u_all,
                          (((1,), (1,)), ((), ())),
                          preferred_element_type=jnp.float32)
    x = pos - neg
    loss = -jnp.log(GAMMA + jax.nn.sigmoid(x))
    out_ref[0, 0] += jnp.sum(loss) * (1.0 / (B * B))


_loss_call = pl.pallas_call(
    _loss_body,
    grid=(B // BLK,),
    in_specs=[
        pl.BlockSpec((G, 128, D), lambda i: (0, 0, 0)),
        pl.BlockSpec((BLK // 128, 128, D), lambda i: (i, 0, 0)),
        pl.BlockSpec((BLK // 128, 128, D), lambda i: (i, 0, 0)),
        pl.BlockSpec((BLK // 128, 128, D), lambda i: (i, 0, 0)),
    ],
    out_specs=pl.BlockSpec((1, 1), lambda i: (0, 0), memory_space=pltpu.SMEM),
    out_shape=jax.ShapeDtypeStruct((1, 1), jnp.float32),
)


def kernel(users, items, neg_items, user_table, item_table):
    users = users.astype(jnp.int32)
    items = items.astype(jnp.int32)
    neg = neg_items.reshape(-1).astype(jnp.int32)
    ut3 = user_table.reshape(-1, G, D)
    it3 = item_table.reshape(-1, G, D)
    g_u, g_p, g_n = _gather3(ut3, it3, users, items, neg)
    out = _loss_call(g_u, g_u, g_p, g_n)
    return out[0, 0]
